# Initial kernel scaffold; baseline (speedup 1.0000x reference)
import jax
import jax.numpy as jnp
from jax import lax
from jax.experimental import pallas as pl
from jax.experimental.pallas import tpu as pltpu
from jax.experimental.pallas import tpu_sc as plsc

VOCAB = 100000
D = 64
B = 1024
L = 200
M = 4
NHOP = 3

TOTAL_ROWS = NHOP * B * L
NW = 32
ROWS_PER_W = TOTAL_ROWS // NW
CHUNK_ROWS = 32
CHUNK_IDX = CHUNK_ROWS * M
N_CHUNKS = ROWS_PER_W // CHUNK_ROWS


def _pool_body(idx_hbm, table_hbm, out_hbm, idx_v, rows_v, pool_v, sem):
    nc = 2
    wid = lax.axis_index("s") * nc + lax.axis_index("c")
    base_row = wid * ROWS_PER_W

    def chunk(i, carry):
        row0 = base_row + i * CHUNK_ROWS
        i0 = row0 * M
        pltpu.sync_copy(idx_hbm.at[pl.ds(i0, CHUNK_IDX)], idx_v)
        pltpu.async_copy(table_hbm.at[idx_v], rows_v, sem).wait()

        def ksum(k, c):
            for d in range(D // 16):
                sl = pl.ds(d * 16, 16)
                pool_v[k, sl] = (rows_v[4 * k, sl] + rows_v[4 * k + 1, sl]
                                 + rows_v[4 * k + 2, sl]
                                 + rows_v[4 * k + 3, sl])
            return c

        lax.fori_loop(0, CHUNK_ROWS, ksum, 0)
        pltpu.sync_copy(pool_v, out_hbm.at[pl.ds(row0, CHUNK_ROWS)])
        return carry

    lax.fori_loop(0, N_CHUNKS, chunk, 0)


def _pool_sc(flat_idx, table):
    f = pl.kernel(
        _pool_body,
        out_type=jax.ShapeDtypeStruct((TOTAL_ROWS, D), jnp.float32),
        mesh=plsc.VectorSubcoreMesh(core_axis_name="c", subcore_axis_name="s"),
        scratch_types=[
            pltpu.VMEM((CHUNK_IDX,), jnp.int32),
            pltpu.VMEM((CHUNK_IDX, D), jnp.float32),
            pltpu.VMEM((CHUNK_ROWS, D), jnp.float32),
            pltpu.SemaphoreType.DMA,
        ],
    )
    return f(flat_idx, table)


BBLK = 32


def _attn_body(q_ref, gp_ref, dh_ref, len_ref, p_ref, soft_ref, logit_ref):
    u = q_ref[...]
    g = gp_ref[...]
    lens = len_ref[...]
    lpos = lax.broadcasted_iota(jnp.int32, (BBLK, L), 1)
    mask = (lpos < lens).astype(jnp.float32)
    base = dh_ref[...] * mask[:, :, None]
    p = p_ref[...]
    gz = g[:, :, None]
    mems = [(p[h] + base) * gz for h in range(NHOP)]

    logits = None
    soft = None
    for h in range(NHOP):
        logits = jnp.sum(mems[h] * u[:, None, :], axis=2)
        mx = jnp.max(logits, axis=1, keepdims=True)
        e = jnp.exp(logits - mx)
        soft = e / jnp.sum(e, axis=1, keepdims=True)
        if h < NHOP - 1:
            u = u + jnp.sum(mems[h + 1] * soft[:, :, None], axis=1)
    soft_ref[...] = soft
    logit_ref[...] = logits


def _attn_tc(q, gp, dh, lens2d, pooled):
    grid = (B // BBLK,)
    out_shape = [
        jax.ShapeDtypeStruct((B, L), jnp.float32),
        jax.ShapeDtypeStruct((B, L), jnp.float32),
    ]
    return pl.pallas_call(
        _attn_body,
        grid=grid,
        in_specs=[
            pl.BlockSpec((BBLK, D), lambda i: (i, 0)),
            pl.BlockSpec((BBLK, L), lambda i: (i, 0)),
            pl.BlockSpec((BBLK, L, D), lambda i: (i, 0, 0)),
            pl.BlockSpec((BBLK, 1), lambda i: (i, 0)),
            pl.BlockSpec((NHOP, BBLK, L, D), lambda i: (0, i, 0, 0)),
        ],
        out_specs=[
            pl.BlockSpec((BBLK, L), lambda i: (i, 0)),
            pl.BlockSpec((BBLK, L), lambda i: (i, 0)),
        ],
        out_shape=out_shape,
    )(q, gp, dh, lens2d, pooled)


def kernel(query_vector, global_pointer, dh_outputs, CS, story, conv_len):
    table = CS.reshape(-1, D)
    offs = (jnp.arange(NHOP, dtype=jnp.int32) * VOCAB)[:, None]
    flat_idx = (story.reshape(1, -1).astype(jnp.int32) + offs).reshape(-1)
    pooled = _pool_sc(flat_idx, table).reshape(NHOP, B, L, D)
    soft, logits = _attn_tc(query_vector, global_pointer, dh_outputs,
                            conv_len.reshape(B, 1).astype(jnp.int32), pooled)
    return (soft, logits)


# speedup vs baseline: 4.1183x; 4.1183x over previous
import jax
import jax.numpy as jnp
from jax import lax
from jax.experimental import pallas as pl
from jax.experimental.pallas import tpu as pltpu
from jax.experimental.pallas import tpu_sc as plsc

VOCAB = 100000
D = 64
B = 1024
L = 200
M = 4
NHOP = 3

TOTAL_ROWS = NHOP * B * L
NW = 32
ROWS_PER_W = TOTAL_ROWS // NW
CHUNK_ROWS = 32
CHUNK_IDX = CHUNK_ROWS * M
N_CHUNKS = ROWS_PER_W // CHUNK_ROWS


def _pool_body(idx_hbm, table_hbm, out_hbm, idx_v, rows_v, pool_v, sem):
    nc = 2
    wid = lax.axis_index("s") * nc + lax.axis_index("c")
    base_row = wid * ROWS_PER_W

    def chunk(i, carry):
        row0 = base_row + i * CHUNK_ROWS
        i0 = row0 * M
        pltpu.sync_copy(idx_hbm.at[pl.ds(i0, CHUNK_IDX)], idx_v)
        pltpu.async_copy(table_hbm.at[idx_v], rows_v, sem).wait()

        def ksum(k, c):
            for d in range(D // 16):
                sl = pl.ds(d * 16, 16)
                pool_v[k, sl] = (rows_v[4 * k, sl] + rows_v[4 * k + 1, sl]
                                 + rows_v[4 * k + 2, sl]
                                 + rows_v[4 * k + 3, sl])
            return c

        lax.fori_loop(0, CHUNK_ROWS, ksum, 0)
        pltpu.sync_copy(pool_v, out_hbm.at[pl.ds(row0, CHUNK_ROWS)])
        return carry

    lax.fori_loop(0, N_CHUNKS, chunk, 0)


def _pool_sc(flat_idx, table):
    f = pl.kernel(
        _pool_body,
        out_type=jax.ShapeDtypeStruct((TOTAL_ROWS, D), jnp.float32),
        mesh=plsc.VectorSubcoreMesh(core_axis_name="c", subcore_axis_name="s",
                                    num_cores=2, num_subcores=16),
        scratch_types=[
            pltpu.VMEM((CHUNK_IDX,), jnp.int32),
            pltpu.VMEM((CHUNK_IDX, D), jnp.float32),
            pltpu.VMEM((CHUNK_ROWS, D), jnp.float32),
            pltpu.SemaphoreType.DMA,
        ],
        compiler_params=pltpu.CompilerParams(use_tc_tiling_on_sc=False),
    )
    return f(flat_idx, table)


BBLK = 32


def _attn_body(q_ref, gp_ref, dh_ref, len_ref, p_ref, soft_ref, logit_ref):
    u = q_ref[...]
    g = gp_ref[...]
    lens = len_ref[...]
    lpos = lax.broadcasted_iota(jnp.int32, (BBLK, L), 1)
    mask = (lpos < lens).astype(jnp.float32)
    base = dh_ref[...] * mask[:, :, None]
    p = p_ref[...]
    gz = g[:, :, None]
    mems = [(p[h] + base) * gz for h in range(NHOP)]

    logits = None
    soft = None
    for h in range(NHOP):
        logits = jnp.sum(mems[h] * u[:, None, :], axis=2)
        mx = jnp.max(logits, axis=1, keepdims=True)
        e = jnp.exp(logits - mx)
        soft = e / jnp.sum(e, axis=1, keepdims=True)
        if h < NHOP - 1:
            u = u + jnp.sum(mems[h + 1] * soft[:, :, None], axis=1)
    soft_ref[...] = soft
    logit_ref[...] = logits


def _attn_tc(q, gp, dh, lens2d, pooled):
    grid = (B // BBLK,)
    out_shape = [
        jax.ShapeDtypeStruct((B, L), jnp.float32),
        jax.ShapeDtypeStruct((B, L), jnp.float32),
    ]
    return pl.pallas_call(
        _attn_body,
        grid=grid,
        in_specs=[
            pl.BlockSpec((BBLK, D), lambda i: (i, 0)),
            pl.BlockSpec((BBLK, L), lambda i: (i, 0)),
            pl.BlockSpec((BBLK, L, D), lambda i: (i, 0, 0)),
            pl.BlockSpec((BBLK, 1), lambda i: (i, 0)),
            pl.BlockSpec((NHOP, BBLK, L, D), lambda i: (0, i, 0, 0)),
        ],
        out_specs=[
            pl.BlockSpec((BBLK, L), lambda i: (i, 0)),
            pl.BlockSpec((BBLK, L), lambda i: (i, 0)),
        ],
        out_shape=out_shape,
    )(q, gp, dh, lens2d, pooled)


def kernel(query_vector, global_pointer, dh_outputs, CS, story, conv_len):
    table = CS.reshape(-1, D)
    offs = (jnp.arange(NHOP, dtype=jnp.int32) * VOCAB)[:, None]
    flat_idx = (story.reshape(1, -1).astype(jnp.int32) + offs).reshape(-1)
    pooled = _pool_sc(flat_idx, table).reshape(NHOP, B, L, D)
    soft, logits = _attn_tc(query_vector, global_pointer, dh_outputs,
                            conv_len.reshape(B, 1).astype(jnp.int32), pooled)
    return (soft, logits)


# speedup vs baseline: 7.2739x; 1.7662x over previous
import jax
import jax.numpy as jnp
from jax import lax
from jax.experimental import pallas as pl
from jax.experimental.pallas import tpu as pltpu
from jax.experimental.pallas import tpu_sc as plsc

VOCAB = 100000
D = 64
B = 1024
L = 200
M = 4
NHOP = 3

TOTAL_ROWS = NHOP * B * L
NW = 32
ROWS_PER_W = TOTAL_ROWS // NW
HOP_ROWS = B * L

G = 128
KG = 4
SC_IDX = G * KG
SC_ROWS = SC_IDX // M
N_SUPER = ROWS_PER_W // SC_ROWS
STORY_COLS = G


def _pool_body(story_hbm, table_hbm, out_hbm, idx_v, rows_v, pool_v,
               sem_i0, sem_i1, sem_g0, sem_g1, sem_w0, sem_w1):
    wid = lax.axis_index("s") * 2 + lax.axis_index("c")
    base_row = wid * ROWS_PER_W
    sem_i = (sem_i0, sem_i1)
    sem_g = (sem_g0, sem_g1)
    sem_w = (sem_w0, sem_w1)

    def chunk_info(s):
        s = jnp.minimum(s, N_SUPER - 1)
        row0 = base_row + s * SC_ROWS
        hop = row0 // HOP_ROWS
        soff = (row0 - hop * HOP_ROWS) // (G // M)
        return row0, hop, soff

    def idx_copy(s, buf):
        _, _, soff = chunk_info(s)
        return pltpu.make_async_copy(
            story_hbm.at[pl.ds(soff, KG)], idx_v.at[buf], sem_i[buf])

    def add_offs(s, buf):
        _, hop, _ = chunk_info(s)
        off = (hop * VOCAB).astype(jnp.int32)
        for j in range(KG):
            for k in range(G // 16):
                sl = pl.ds(k * 16, 16)
                idx_v[buf, j, sl] = idx_v[buf, j, sl] + off

    def gather_copies(buf):
        return [pltpu.make_async_copy(
                    table_hbm.at[idx_v.at[buf, j]],
                    rows_v.at[buf, pl.ds(j * G, G)], sem_g[buf])
                for j in range(KG)]

    def write_copy(s, buf):
        row0, _, _ = chunk_info(s)
        return pltpu.make_async_copy(
            pool_v.at[buf], out_hbm.at[pl.ds(row0, SC_ROWS)], sem_w[buf])

    def quadsum(buf):
        def body(k, c):
            for r in range(2):
                kk = 2 * k + r
                for d in range(D // 16):
                    sl = pl.ds(d * 16, 16)
                    pool_v[buf, kk, sl] = (
                        rows_v[buf, 4 * kk, sl] + rows_v[buf, 4 * kk + 1, sl]
                        + rows_v[buf, 4 * kk + 2, sl]
                        + rows_v[buf, 4 * kk + 3, sl])
            return c
        lax.fori_loop(0, SC_ROWS // 2, body, 0)

    idx_copy(0, 0).start()
    idx_copy(1, 1).start()
    idx_copy(0, 0).wait()
    add_offs(0, 0)
    for c in gather_copies(0):
        c.start()

    def phase(s, buf):
        nxt = 1 - buf
        idx_copy(s + 1, nxt).wait()
        add_offs(s + 1, nxt)
        for c in gather_copies(nxt):
            c.start()
        for c in gather_copies(buf):
            c.wait()
        idx_copy(s + 2, buf).start()

        @pl.when(s >= 2)
        def _():
            write_copy(s - 2, buf).wait()

        quadsum(buf)
        write_copy(s, buf).start()

    def loop_body(t, c):
        phase(2 * t, 0)
        phase(2 * t + 1, 1)
        return c
    lax.fori_loop(0, N_SUPER // 2, loop_body, 0)

    for c in gather_copies(0):
        c.wait()
    idx_copy(N_SUPER + 1, 1).wait()
    write_copy(N_SUPER - 2, 0).wait()
    write_copy(N_SUPER - 1, 1).wait()


def _pool_sc(story2d, table):
    f = pl.kernel(
        _pool_body,
        out_type=jax.ShapeDtypeStruct((TOTAL_ROWS, D), jnp.float32),
        mesh=plsc.VectorSubcoreMesh(core_axis_name="c", subcore_axis_name="s",
                                    num_cores=2, num_subcores=16),
        scratch_types=[
            pltpu.VMEM((2, KG, G), jnp.int32),
            pltpu.VMEM((2, SC_IDX, D), jnp.float32),
            pltpu.VMEM((2, SC_ROWS, D), jnp.float32),
            pltpu.SemaphoreType.DMA,
            pltpu.SemaphoreType.DMA,
            pltpu.SemaphoreType.DMA,
            pltpu.SemaphoreType.DMA,
            pltpu.SemaphoreType.DMA,
            pltpu.SemaphoreType.DMA,
        ],
        compiler_params=pltpu.CompilerParams(use_tc_tiling_on_sc=False),
    )
    return f(story2d, table)


BBLK = 32
LCH = 40
NCH = L // LCH


def _attn_body(q_ref, gp_ref, dh_ref, len_ref, p_ref, soft_ref, logit_ref):
    u = q_ref[...]
    lens = len_ref[...]

    def mem_slice(h, c):
        lsl = pl.ds(c * LCH, LCH)
        lpos = lax.broadcasted_iota(jnp.int32, (BBLK, LCH), 1) + c * LCH
        mask = (lpos < lens).astype(jnp.float32)
        g = gp_ref[:, lsl]
        return ((p_ref[h, :, lsl, :] + dh_ref[:, lsl, :] * mask[:, :, None])
                * g[:, :, None])

    logits = None
    soft = None
    for h in range(NHOP):
        parts = [jnp.sum(mem_slice(h, c) * u[:, None, :], axis=2)
                 for c in range(NCH)]
        logits = jnp.concatenate(parts, axis=1)
        mx = jnp.max(logits, axis=1, keepdims=True)
        e = jnp.exp(logits - mx)
        soft = e / jnp.sum(e, axis=1, keepdims=True)
        if h < NHOP - 1:
            o = jnp.zeros((BBLK, D), jnp.float32)
            for c in range(NCH):
                w = soft[:, c * LCH:(c + 1) * LCH, None]
                o = o + jnp.sum(mem_slice(h + 1, c) * w, axis=1)
            u = u + o
    soft_ref[...] = soft
    logit_ref[...] = logits


def _attn_tc(q, gp, dh, lens2d, pooled):
    grid = (B // BBLK,)
    out_shape = [
        jax.ShapeDtypeStruct((B, L), jnp.float32),
        jax.ShapeDtypeStruct((B, L), jnp.float32),
    ]
    return pl.pallas_call(
        _attn_body,
        grid=grid,
        in_specs=[
            pl.BlockSpec((BBLK, D), lambda i: (i, 0)),
            pl.BlockSpec((BBLK, L), lambda i: (i, 0)),
            pl.BlockSpec((BBLK, L, D), lambda i: (i, 0, 0)),
            pl.BlockSpec((BBLK, 1), lambda i: (i, 0)),
            pl.BlockSpec((NHOP, BBLK, L, D), lambda i: (0, i, 0, 0)),
        ],
        out_specs=[
            pl.BlockSpec((BBLK, L), lambda i: (i, 0)),
            pl.BlockSpec((BBLK, L), lambda i: (i, 0)),
        ],
        out_shape=out_shape,
    )(q, gp, dh, lens2d, pooled)


def kernel(query_vector, global_pointer, dh_outputs, CS, story, conv_len):
    table = CS.reshape(-1, D)
    story2d = story.reshape(-1, STORY_COLS).astype(jnp.int32)
    pooled = _pool_sc(story2d, table).reshape(NHOP, B, L, D)
    soft, logits = _attn_tc(query_vector, global_pointer, dh_outputs,
                            conv_len.reshape(B, 1).astype(jnp.int32), pooled)
    return (soft, logits)
